# Initial kernel scaffold; baseline (speedup 1.0000x reference)
import jax
import jax.numpy as jnp
from jax.experimental import pallas as pl


def kernel(x, edge_attr, node_types, W, b):
    raise NotImplementedError("write your pallas kernel here")


# speedup vs baseline: 1.2110x; 1.2110x over previous
import functools

import jax
import jax.numpy as jnp
from jax import lax
from jax.experimental import pallas as pl
from jax.experimental.pallas import tpu as pltpu
from jax.experimental.pallas import tpu_sc as plsc

N_TYPES = 17
D_HALF = 512
TM = 256
NB = 216
MP = NB * TM

_NC = 2
_NS = 16
_NW = _NC * _NS

_CH_IN = 96
_N_PAD = 50176
_CH_OUT = 112


def _row_gather(table, idx, n_rows, chunk):
    per_w = n_rows // _NW
    iters = per_w // chunk
    mesh = plsc.VectorSubcoreMesh(core_axis_name="c", subcore_axis_name="s")

    @functools.partial(
        pl.kernel,
        mesh=mesh,
        out_type=jax.ShapeDtypeStruct((n_rows, D_HALF), jnp.float32),
        scratch_types=[
            pltpu.VMEM((chunk,), jnp.int32),
            pltpu.VMEM((chunk, D_HALF), jnp.float32),
            pltpu.SemaphoreType.DMA,
        ],
    )
    def gather_kernel(table_hbm, idx_hbm, out_hbm, idx_v, rows_v, sem):
        wid = lax.axis_index("s") * _NC + lax.axis_index("c")
        base = wid * per_w

        def body(c, _):
            off = base + c * chunk
            pltpu.sync_copy(idx_hbm.at[pl.ds(off, chunk)], idx_v)
            pltpu.async_copy(table_hbm.at[idx_v], rows_v, sem).wait()
            pltpu.sync_copy(rows_v, out_hbm.at[pl.ds(off, chunk)])
            return ()

        lax.fori_loop(0, iters, body, ())

    return gather_kernel(table, idx)


def _grouped_matmul(sx, se, block_type, W, b):
    def mm_kernel(bt_ref, a1_ref, a2_ref, w_ref, b_ref, o_ref):
        w = w_ref[0]
        dn = (((1,), (1,)), ((), ()))
        acc = lax.dot_general(a1_ref[...], w[:, :D_HALF], dn,
                              preferred_element_type=jnp.float32)
        acc = acc + lax.dot_general(a2_ref[...], w[:, D_HALF:], dn,
                                    preferred_element_type=jnp.float32)
        o_ref[...] = jnp.maximum(acc + b_ref[0], 0.0)

    grid_spec = pltpu.PrefetchScalarGridSpec(
        num_scalar_prefetch=1,
        grid=(NB,),
        in_specs=[
            pl.BlockSpec((TM, D_HALF), lambda i, bt: (i, 0)),
            pl.BlockSpec((TM, D_HALF), lambda i, bt: (i, 0)),
            pl.BlockSpec((1, D_HALF, 2 * D_HALF), lambda i, bt: (bt[i], 0, 0)),
            pl.BlockSpec((1, 1, D_HALF), lambda i, bt: (bt[i], 0, 0)),
        ],
        out_specs=pl.BlockSpec((TM, D_HALF), lambda i, bt: (i, 0)),
    )
    return pl.pallas_call(
        mm_kernel,
        grid_spec=grid_spec,
        out_shape=jax.ShapeDtypeStruct((MP, D_HALF), jnp.float32),
    )(block_type, sx, se, W, b.reshape(N_TYPES, 1, D_HALF))


def kernel(x, edge_attr, node_types, W, b):
    n = x.shape[0]
    t = node_types.astype(jnp.int32)

    counts = jnp.bincount(t, length=N_TYPES)
    padded = ((counts + TM - 1) // TM) * TM
    pstart = jnp.cumsum(padded) - padded
    ustart = jnp.cumsum(counts) - counts
    order = jnp.argsort(t)
    st = t[order]
    slot = pstart[st] + (jnp.arange(n, dtype=jnp.int32) - ustart[st])
    src = jnp.zeros((MP,), jnp.int32).at[slot].set(order.astype(jnp.int32))
    inv = jnp.zeros((n,), jnp.int32).at[order].set(slot.astype(jnp.int32))
    inv_pad = jnp.zeros((_N_PAD,), jnp.int32).at[:n].set(inv)
    bend = jnp.cumsum(padded) // TM
    block_type = jnp.minimum(
        jnp.searchsorted(bend, jnp.arange(NB, dtype=jnp.int32), side="right"),
        N_TYPES - 1,
    ).astype(jnp.int32)

    sx = _row_gather(x, src, MP, _CH_IN)
    se = _row_gather(edge_attr, src, MP, _CH_IN)

    out_sorted = _grouped_matmul(sx, se, block_type, W, b)

    out_pad = _row_gather(out_sorted, inv_pad, _N_PAD, _CH_OUT)
    return out_pad[:n]


# speedup vs baseline: 1.4956x; 1.2350x over previous
import functools

import jax
import jax.numpy as jnp
from jax import lax
from jax.experimental import pallas as pl
from jax.experimental.pallas import tpu as pltpu
from jax.experimental.pallas import tpu_sc as plsc

N_TYPES = 17
D_HALF = 512
TM = 256
NB = 216
MP = NB * TM

_NC = 2
_NS = 16
_NW = _NC * _NS

_CH_IN = 96
_N_PAD = 50176
_CH_OUT = 112

_RANK_S = 500


def _gather_xe(x, edge_attr, idx):
    per_w = MP // _NW
    iters = per_w // _CH_IN
    mesh = plsc.VectorSubcoreMesh(core_axis_name="c", subcore_axis_name="s")

    @functools.partial(
        pl.kernel,
        mesh=mesh,
        out_type=(
            jax.ShapeDtypeStruct((MP, D_HALF), jnp.float32),
            jax.ShapeDtypeStruct((MP, D_HALF), jnp.float32),
        ),
        scratch_types=[
            pltpu.VMEM((per_w,), jnp.int32),
            pltpu.VMEM((_CH_IN, D_HALF), jnp.float32),
            pltpu.VMEM((_CH_IN, D_HALF), jnp.float32),
            pltpu.SemaphoreType.DMA,
        ],
    )
    def gather_kernel(x_hbm, e_hbm, idx_hbm, sx_hbm, se_hbm, idx_v, rx, re, sem):
        wid = lax.axis_index("s") * _NC + lax.axis_index("c")
        base = wid * per_w
        pltpu.sync_copy(idx_hbm.at[pl.ds(base, per_w)], idx_v)

        def body(c, _):
            off = base + c * _CH_IN
            ids = idx_v.at[pl.ds(c * _CH_IN, _CH_IN)]
            cx = pltpu.async_copy(x_hbm.at[ids], rx, sem)
            ce = pltpu.async_copy(e_hbm.at[ids], re, sem)
            cx.wait()
            ce.wait()
            pltpu.sync_copy(rx, sx_hbm.at[pl.ds(off, _CH_IN)])
            pltpu.sync_copy(re, se_hbm.at[pl.ds(off, _CH_IN)])
            return ()

        lax.fori_loop(0, iters, body, ())

    return gather_kernel(x, edge_attr, idx)


def _gather_out(table, idx):
    per_w = _N_PAD // _NW
    iters = per_w // _CH_OUT
    mesh = plsc.VectorSubcoreMesh(core_axis_name="c", subcore_axis_name="s")

    @functools.partial(
        pl.kernel,
        mesh=mesh,
        out_type=jax.ShapeDtypeStruct((_N_PAD, D_HALF), jnp.float32),
        scratch_types=[
            pltpu.VMEM((per_w,), jnp.int32),
            pltpu.VMEM((_CH_OUT, D_HALF), jnp.float32),
            pltpu.SemaphoreType.DMA,
        ],
    )
    def gather_kernel(table_hbm, idx_hbm, out_hbm, idx_v, rows_v, sem):
        wid = lax.axis_index("s") * _NC + lax.axis_index("c")
        base = wid * per_w
        pltpu.sync_copy(idx_hbm.at[pl.ds(base, per_w)], idx_v)

        def body(c, _):
            off = base + c * _CH_OUT
            ids = idx_v.at[pl.ds(c * _CH_OUT, _CH_OUT)]
            pltpu.async_copy(table_hbm.at[ids], rows_v, sem).wait()
            pltpu.sync_copy(rows_v, out_hbm.at[pl.ds(off, _CH_OUT)])
            return ()

        lax.fori_loop(0, iters, body, ())

    return gather_kernel(table, idx)


def _grouped_matmul(sx, se, block_type, W16, b):
    def mm_kernel(bt_ref, a1_ref, a2_ref, w_ref, b_ref, o_ref):
        w = w_ref[0]
        a1 = a1_ref[...].astype(jnp.bfloat16)
        a2 = a2_ref[...].astype(jnp.bfloat16)
        dn = (((1,), (1,)), ((), ()))
        acc = lax.dot_general(a1, w[:, :D_HALF], dn,
                              preferred_element_type=jnp.float32)
        acc = acc + lax.dot_general(a2, w[:, D_HALF:], dn,
                                    preferred_element_type=jnp.float32)
        o_ref[...] = jnp.maximum(acc + b_ref[0], 0.0)

    grid_spec = pltpu.PrefetchScalarGridSpec(
        num_scalar_prefetch=1,
        grid=(NB,),
        in_specs=[
            pl.BlockSpec((TM, D_HALF), lambda i, bt: (i, 0)),
            pl.BlockSpec((TM, D_HALF), lambda i, bt: (i, 0)),
            pl.BlockSpec((1, D_HALF, 2 * D_HALF), lambda i, bt: (bt[i], 0, 0)),
            pl.BlockSpec((1, 1, D_HALF), lambda i, bt: (bt[i], 0, 0)),
        ],
        out_specs=pl.BlockSpec((TM, D_HALF), lambda i, bt: (i, 0)),
    )
    return pl.pallas_call(
        mm_kernel,
        grid_spec=grid_spec,
        out_shape=jax.ShapeDtypeStruct((MP, D_HALF), jnp.float32),
    )(block_type, sx, se, W16, b.reshape(N_TYPES, 1, D_HALF))


def kernel(x, edge_attr, node_types, W, b):
    n = x.shape[0]
    t = node_types.astype(jnp.int32)

    nchunks = n // _RANK_S
    oh = (t[:, None] == jnp.arange(N_TYPES, dtype=jnp.int32)[None, :])
    ohf = oh.astype(jnp.float32).reshape(nchunks, _RANK_S, N_TYPES)
    tri = jnp.tril(jnp.ones((_RANK_S, _RANK_S), jnp.float32), k=-1)
    local_rank = lax.dot_general(
        tri, ohf, (((1,), (1,)), ((), ())),
        precision=lax.Precision.HIGHEST).transpose(1, 0, 2)
    chunk_cnt = ohf.sum(axis=1)
    chunk_base = jnp.cumsum(chunk_cnt, axis=0) - chunk_cnt
    counts = chunk_cnt.sum(axis=0).astype(jnp.int32)
    padded = ((counts + TM - 1) // TM) * TM
    pstart = (jnp.cumsum(padded) - padded).astype(jnp.float32)
    slot_f = ((pstart[None, None, :] + chunk_base[:, None, :] + local_rank)
              * ohf).sum(axis=-1)
    slot = slot_f.reshape(n).astype(jnp.int32)
    src = jnp.zeros((MP,), jnp.int32).at[slot].set(
        jnp.arange(n, dtype=jnp.int32))
    inv_pad = jnp.zeros((_N_PAD,), jnp.int32).at[:n].set(slot)
    bend = (jnp.cumsum(padded) // TM).astype(jnp.int32)
    block_type = jnp.minimum(
        jnp.searchsorted(bend, jnp.arange(NB, dtype=jnp.int32), side="right"),
        N_TYPES - 1,
    ).astype(jnp.int32)

    sx, se = _gather_xe(x, edge_attr, src)

    out_sorted = _grouped_matmul(sx, se, block_type, W.astype(jnp.bfloat16), b)

    out_pad = _gather_out(out_sorted, inv_pad)
    return out_pad[:n]


# speedup vs baseline: 3.4591x; 2.3129x over previous
import functools

import jax
import jax.numpy as jnp
from jax import lax
from jax.experimental import pallas as pl
from jax.experimental.pallas import tpu as pltpu
from jax.experimental.pallas import tpu_sc as plsc

N_TYPES = 17
D_HALF = 512
TM = 256
NB = 216
MP = NB * TM

_NC = 2
_NS = 16
_NW = _NC * _NS

_CH = 112
_CPW = 14

_RANK_S = 500


def _dispatch(x, edge_attr, idx2d):
    n = x.shape[0]
    mesh = plsc.VectorSubcoreMesh(core_axis_name="c", subcore_axis_name="s")

    @functools.partial(
        pl.kernel,
        mesh=mesh,
        out_type=(
            jax.ShapeDtypeStruct((MP, D_HALF), jnp.float32),
            jax.ShapeDtypeStruct((MP, D_HALF), jnp.float32),
        ),
        scratch_types=[
            pltpu.VMEM((_CH,), jnp.int32),
            pltpu.VMEM((_CH, D_HALF), jnp.float32),
            pltpu.VMEM((_CH, D_HALF), jnp.float32),
            pltpu.SemaphoreType.DMA,
        ],
    )
    def dispatch_kernel(x_hbm, e_hbm, idx_hbm, sx_hbm, se_hbm, idx_v, rx, re, sem):
        wid = lax.axis_index("s") * _NC + lax.axis_index("c")

        def body(c, _):
            q = wid * _CPW + c
            start = jnp.minimum(q * _CH, n - _CH)
            pltpu.sync_copy(idx_hbm.at[q], idx_v)
            pltpu.sync_copy(x_hbm.at[pl.ds(start, _CH)], rx)
            pltpu.sync_copy(e_hbm.at[pl.ds(start, _CH)], re)
            cx = pltpu.async_copy(rx, sx_hbm.at[idx_v], sem)
            ce = pltpu.async_copy(re, se_hbm.at[idx_v], sem)
            cx.wait()
            ce.wait()
            return ()

        lax.fori_loop(0, _CPW, body, ())

    return dispatch_kernel(x, edge_attr, idx2d)


def _collect(table, idx2d, n):
    mesh = plsc.VectorSubcoreMesh(core_axis_name="c", subcore_axis_name="s")

    @functools.partial(
        pl.kernel,
        mesh=mesh,
        out_type=jax.ShapeDtypeStruct((n, D_HALF), jnp.float32),
        scratch_types=[
            pltpu.VMEM((_CH,), jnp.int32),
            pltpu.VMEM((_CH, D_HALF), jnp.float32),
            pltpu.SemaphoreType.DMA,
        ],
    )
    def collect_kernel(table_hbm, idx_hbm, out_hbm, idx_v, rows_v, sem):
        wid = lax.axis_index("s") * _NC + lax.axis_index("c")

        def body(c, _):
            q = wid * _CPW + c
            start = jnp.minimum(q * _CH, n - _CH)
            pltpu.sync_copy(idx_hbm.at[q], idx_v)
            pltpu.async_copy(table_hbm.at[idx_v], rows_v, sem).wait()
            pltpu.sync_copy(rows_v, out_hbm.at[pl.ds(start, _CH)])
            return ()

        lax.fori_loop(0, _CPW, body, ())

    return collect_kernel(table, idx2d)


def _grouped_matmul(sx, se, block_type, W16, b):
    def mm_kernel(bt_ref, a1_ref, a2_ref, w_ref, b_ref, o_ref):
        w = w_ref[0]
        a1 = a1_ref[...].astype(jnp.bfloat16)
        a2 = a2_ref[...].astype(jnp.bfloat16)
        dn = (((1,), (1,)), ((), ()))
        acc = lax.dot_general(a1, w[:, :D_HALF], dn,
                              preferred_element_type=jnp.float32)
        acc = acc + lax.dot_general(a2, w[:, D_HALF:], dn,
                                    preferred_element_type=jnp.float32)
        o_ref[...] = jnp.maximum(acc + b_ref[0], 0.0)

    grid_spec = pltpu.PrefetchScalarGridSpec(
        num_scalar_prefetch=1,
        grid=(NB,),
        in_specs=[
            pl.BlockSpec((TM, D_HALF), lambda i, bt: (i, 0)),
            pl.BlockSpec((TM, D_HALF), lambda i, bt: (i, 0)),
            pl.BlockSpec((1, D_HALF, 2 * D_HALF), lambda i, bt: (bt[i], 0, 0)),
            pl.BlockSpec((1, 1, D_HALF), lambda i, bt: (bt[i], 0, 0)),
        ],
        out_specs=pl.BlockSpec((TM, D_HALF), lambda i, bt: (i, 0)),
    )
    return pl.pallas_call(
        mm_kernel,
        grid_spec=grid_spec,
        out_shape=jax.ShapeDtypeStruct((MP, D_HALF), jnp.float32),
    )(block_type, sx, se, W16, b.reshape(N_TYPES, 1, D_HALF))


def kernel(x, edge_attr, node_types, W, b):
    n = x.shape[0]
    t = node_types.astype(jnp.int32)

    nchunks = n // _RANK_S
    oh = (t[:, None] == jnp.arange(N_TYPES, dtype=jnp.int32)[None, :])
    ohf = oh.astype(jnp.float32).reshape(nchunks, _RANK_S, N_TYPES)
    tri = jnp.tril(jnp.ones((_RANK_S, _RANK_S), jnp.float32), k=-1)
    local_rank = lax.dot_general(
        tri, ohf, (((1,), (1,)), ((), ())),
        precision=lax.Precision.HIGHEST).transpose(1, 0, 2)
    chunk_cnt = ohf.sum(axis=1)
    chunk_base = jnp.cumsum(chunk_cnt, axis=0) - chunk_cnt
    counts = chunk_cnt.sum(axis=0).astype(jnp.int32)
    padded = ((counts + TM - 1) // TM) * TM
    pstart = (jnp.cumsum(padded) - padded).astype(jnp.float32)
    slot_f = ((pstart[None, None, :] + chunk_base[:, None, :] + local_rank)
              * ohf).sum(axis=-1)
    slot = slot_f.reshape(n).astype(jnp.int32)
    starts = jnp.minimum(
        jnp.arange(_NW * _CPW, dtype=jnp.int32) * _CH, n - _CH)
    idx2d = slot[starts[:, None] + jnp.arange(_CH, dtype=jnp.int32)[None, :]]
    bend = (jnp.cumsum(padded) // TM).astype(jnp.int32)
    block_type = jnp.minimum(
        jnp.searchsorted(bend, jnp.arange(NB, dtype=jnp.int32), side="right"),
        N_TYPES - 1,
    ).astype(jnp.int32)

    sx, se = _dispatch(x, edge_attr, idx2d)

    out_sorted = _grouped_matmul(sx, se, block_type, W.astype(jnp.bfloat16), b)

    return _collect(out_sorted, idx2d, n)


# speedup vs baseline: 3.5418x; 1.0239x over previous
import functools

import jax
import jax.numpy as jnp
from jax import lax
from jax.experimental import pallas as pl
from jax.experimental.pallas import tpu as pltpu
from jax.experimental.pallas import tpu_sc as plsc

N_TYPES = 17
D_HALF = 512
TM = 256
NB = 216
MP = NB * TM

_NC = 2
_NS = 16
_NW = _NC * _NS

_CH = 112
_CPW = 14

_RANK_S = 128


def _dispatch(x, edge_attr, slot):
    n = x.shape[0]
    mesh = plsc.VectorSubcoreMesh(core_axis_name="c", subcore_axis_name="s")

    @functools.partial(
        pl.kernel,
        mesh=mesh,
        out_type=(
            jax.ShapeDtypeStruct((MP, D_HALF), jnp.float32),
            jax.ShapeDtypeStruct((MP, D_HALF), jnp.float32),
        ),
        scratch_types=[
            pltpu.VMEM((_CH,), jnp.int32),
            pltpu.VMEM((_CH, D_HALF), jnp.float32),
            pltpu.VMEM((_CH, D_HALF), jnp.float32),
            pltpu.SemaphoreType.DMA,
        ],
    )
    def dispatch_kernel(x_hbm, e_hbm, idx_hbm, sx_hbm, se_hbm, idx_v, rx, re, sem):
        wid = lax.axis_index("s") * _NC + lax.axis_index("c")

        def body(c, _):
            q = wid * _CPW + c
            start = jnp.minimum(q * _CH, n - _CH)
            pltpu.sync_copy(idx_hbm.at[pl.ds(start, _CH)], idx_v)
            pltpu.sync_copy(x_hbm.at[pl.ds(start, _CH)], rx)
            pltpu.sync_copy(e_hbm.at[pl.ds(start, _CH)], re)
            cx = pltpu.async_copy(rx, sx_hbm.at[idx_v], sem)
            ce = pltpu.async_copy(re, se_hbm.at[idx_v], sem)
            cx.wait()
            ce.wait()
            return ()

        lax.fori_loop(0, _CPW, body, ())

    return dispatch_kernel(x, edge_attr, slot)


def _collect(table, slot, n):
    mesh = plsc.VectorSubcoreMesh(core_axis_name="c", subcore_axis_name="s")

    @functools.partial(
        pl.kernel,
        mesh=mesh,
        out_type=jax.ShapeDtypeStruct((n, D_HALF), jnp.float32),
        scratch_types=[
            pltpu.VMEM((_CH,), jnp.int32),
            pltpu.VMEM((_CH, D_HALF), jnp.float32),
            pltpu.SemaphoreType.DMA,
        ],
    )
    def collect_kernel(table_hbm, idx_hbm, out_hbm, idx_v, rows_v, sem):
        wid = lax.axis_index("s") * _NC + lax.axis_index("c")

        def body(c, _):
            q = wid * _CPW + c
            start = jnp.minimum(q * _CH, n - _CH)
            pltpu.sync_copy(idx_hbm.at[pl.ds(start, _CH)], idx_v)
            pltpu.async_copy(table_hbm.at[idx_v], rows_v, sem).wait()
            pltpu.sync_copy(rows_v, out_hbm.at[pl.ds(start, _CH)])
            return ()

        lax.fori_loop(0, _CPW, body, ())

    return collect_kernel(table, slot)


def _grouped_matmul(sx, se, block_type, W16, b):
    def mm_kernel(bt_ref, a1_ref, a2_ref, w_ref, b_ref, o_ref):
        w = w_ref[0]
        a1 = a1_ref[...].astype(jnp.bfloat16)
        a2 = a2_ref[...].astype(jnp.bfloat16)
        dn = (((1,), (1,)), ((), ()))
        acc = lax.dot_general(a1, w[:, :D_HALF], dn,
                              preferred_element_type=jnp.float32)
        acc = acc + lax.dot_general(a2, w[:, D_HALF:], dn,
                                    preferred_element_type=jnp.float32)
        o_ref[...] = jnp.maximum(acc + b_ref[0], 0.0)

    grid_spec = pltpu.PrefetchScalarGridSpec(
        num_scalar_prefetch=1,
        grid=(NB,),
        in_specs=[
            pl.BlockSpec((TM, D_HALF), lambda i, bt: (i, 0)),
            pl.BlockSpec((TM, D_HALF), lambda i, bt: (i, 0)),
            pl.BlockSpec((1, D_HALF, 2 * D_HALF), lambda i, bt: (bt[i], 0, 0)),
            pl.BlockSpec((1, 1, D_HALF), lambda i, bt: (bt[i], 0, 0)),
        ],
        out_specs=pl.BlockSpec((TM, D_HALF), lambda i, bt: (i, 0)),
    )
    return pl.pallas_call(
        mm_kernel,
        grid_spec=grid_spec,
        out_shape=jax.ShapeDtypeStruct((MP, D_HALF), jnp.float32),
    )(block_type, sx, se, W16, b.reshape(N_TYPES, 1, D_HALF))


def kernel(x, edge_attr, node_types, W, b):
    n = x.shape[0]
    t = node_types.astype(jnp.int32)

    nchunks = -(-n // _RANK_S)
    np2 = nchunks * _RANK_S
    t_pad = jnp.pad(t, (0, np2 - n), constant_values=N_TYPES)
    ohf = (t_pad[None, :] == jnp.arange(N_TYPES, dtype=jnp.int32)[:, None]
           ).astype(jnp.float32).reshape(N_TYPES, nchunks, _RANK_S)
    ar = jnp.arange(_RANK_S, dtype=jnp.int32)
    tri = (ar[:, None] < ar[None, :]).astype(jnp.float32)
    local_rank = lax.dot_general(
        ohf, tri, (((2,), (0,)), ((), ())),
        precision=lax.Precision.HIGHEST)
    chunk_cnt = ohf.sum(axis=2)
    chunk_base = jnp.cumsum(chunk_cnt, axis=1) - chunk_cnt
    counts = chunk_cnt.sum(axis=1).astype(jnp.int32)
    padded = ((counts + TM - 1) // TM) * TM
    pstart = (jnp.cumsum(padded) - padded).astype(jnp.float32)
    slot_f = ((local_rank + chunk_base[:, :, None]
               + pstart[:, None, None]) * ohf).sum(axis=0)
    slot = slot_f.reshape(np2).astype(jnp.int32)
    bend = (jnp.cumsum(padded) // TM).astype(jnp.int32)
    block_type = jnp.minimum(
        jnp.searchsorted(bend, jnp.arange(NB, dtype=jnp.int32), side="right"),
        N_TYPES - 1,
    ).astype(jnp.int32)

    sx, se = _dispatch(x, edge_attr, slot)

    out_sorted = _grouped_matmul(sx, se, block_type, W.astype(jnp.bfloat16), b)

    return _collect(out_sorted, slot, n)


# speedup vs baseline: 3.6650x; 1.0348x over previous
import functools

import jax
import jax.numpy as jnp
from jax import lax
from jax.experimental import pallas as pl
from jax.experimental.pallas import tpu as pltpu
from jax.experimental.pallas import tpu_sc as plsc

N_TYPES = 17
D_HALF = 512
TM = 256
NB = 216
MP = NB * TM

_NC = 2
_NS = 16
_NW = _NC * _NS

_CHD = 56
_CPWD = 28
_CHC = 112
_CPWC = 14

_RANK_S = 128


def _dispatch(x, edge_attr, slot):
    n = x.shape[0]
    mesh = plsc.VectorSubcoreMesh(core_axis_name="c", subcore_axis_name="s")

    @functools.partial(
        pl.kernel,
        mesh=mesh,
        out_type=(
            jax.ShapeDtypeStruct((MP, D_HALF), jnp.float32),
            jax.ShapeDtypeStruct((MP, D_HALF), jnp.float32),
        ),
        scratch_types=[
            pltpu.VMEM((_CHD,), jnp.int32),
            pltpu.VMEM((_CHD,), jnp.int32),
            pltpu.VMEM((_CHD, D_HALF), jnp.float32),
            pltpu.VMEM((_CHD, D_HALF), jnp.float32),
            pltpu.VMEM((_CHD, D_HALF), jnp.float32),
            pltpu.VMEM((_CHD, D_HALF), jnp.float32),
            pltpu.SemaphoreType.DMA,
            pltpu.SemaphoreType.DMA,
            pltpu.SemaphoreType.DMA,
            pltpu.SemaphoreType.DMA,
        ],
    )
    def dispatch_kernel(x_hbm, e_hbm, idx_hbm, sx_hbm, se_hbm,
                        iv0, iv1, rx0, rx1, re0, re1, sx0, sx1, se0, se1):
        wid = lax.axis_index("s") * _NC + lax.axis_index("c")
        iv, rx, re = [iv0, iv1], [rx0, rx1], [re0, re1]
        ssx, sse = [sx0, sx1], [se0, se1]

        def step(k, _):
            for i in (0, 1):
                q = wid * _CPWD + 2 * k + i
                start = jnp.minimum(q * _CHD, n - _CHD)

                @pl.when(k > 0)
                def _():
                    pltpu.make_async_copy(rx[i], sx_hbm.at[iv[i]], ssx[i]).wait()
                    pltpu.make_async_copy(re[i], se_hbm.at[iv[i]], sse[i]).wait()

                pltpu.sync_copy(idx_hbm.at[pl.ds(start, _CHD)], iv[i])
                pltpu.sync_copy(x_hbm.at[pl.ds(start, _CHD)], rx[i])
                pltpu.sync_copy(e_hbm.at[pl.ds(start, _CHD)], re[i])
                pltpu.async_copy(rx[i], sx_hbm.at[iv[i]], ssx[i])
                pltpu.async_copy(re[i], se_hbm.at[iv[i]], sse[i])
            return ()

        lax.fori_loop(0, _CPWD // 2, step, ())
        for i in (0, 1):
            pltpu.make_async_copy(rx[i], sx_hbm.at[iv[i]], ssx[i]).wait()
            pltpu.make_async_copy(re[i], se_hbm.at[iv[i]], sse[i]).wait()

    return dispatch_kernel(x, edge_attr, slot)


def _collect(table, slot, n):
    mesh = plsc.VectorSubcoreMesh(core_axis_name="c", subcore_axis_name="s")

    @functools.partial(
        pl.kernel,
        mesh=mesh,
        out_type=jax.ShapeDtypeStruct((n, D_HALF), jnp.float32),
        scratch_types=[
            pltpu.VMEM((_CHC,), jnp.int32),
            pltpu.VMEM((_CHC,), jnp.int32),
            pltpu.VMEM((_CHC, D_HALF), jnp.float32),
            pltpu.VMEM((_CHC, D_HALF), jnp.float32),
            pltpu.SemaphoreType.DMA,
            pltpu.SemaphoreType.DMA,
            pltpu.SemaphoreType.DMA,
            pltpu.SemaphoreType.DMA,
        ],
    )
    def collect_kernel(table_hbm, idx_hbm, out_hbm,
                       iv0, iv1, rv0, rv1, g0, g1, w0, w1):
        wid = lax.axis_index("s") * _NC + lax.axis_index("c")
        iv, rv = [iv0, iv1], [rv0, rv1]
        gs, ws = [g0, g1], [w0, w1]

        def chunk_start(k, i):
            q = wid * _CPWC + 2 * k + i
            return jnp.minimum(q * _CHC, n - _CHC)

        def step(k, _):
            for i in (0, 1):
                start = chunk_start(k, i)

                @pl.when(k > 0)
                def _():
                    pltpu.make_async_copy(
                        rv[i], out_hbm.at[pl.ds(start, _CHC)], ws[i]).wait()

                pltpu.sync_copy(idx_hbm.at[pl.ds(start, _CHC)], iv[i])
                pltpu.async_copy(table_hbm.at[iv[i]], rv[i], gs[i])
            for i in (0, 1):
                start = chunk_start(k, i)
                pltpu.make_async_copy(table_hbm.at[iv[i]], rv[i], gs[i]).wait()
                pltpu.async_copy(rv[i], out_hbm.at[pl.ds(start, _CHC)], ws[i])
            return ()

        lax.fori_loop(0, _CPWC // 2, step, ())
        for i in (0, 1):
            pltpu.make_async_copy(
                rv[i], out_hbm.at[pl.ds(chunk_start(_CPWC // 2 - 1, i), _CHC)],
                ws[i]).wait()

    return collect_kernel(table, slot)


def _grouped_matmul(sx, se, block_type, W16, b):
    def mm_kernel(bt_ref, a1_ref, a2_ref, w_ref, b_ref, o_ref):
        w = w_ref[0]
        a1 = a1_ref[...].astype(jnp.bfloat16)
        a2 = a2_ref[...].astype(jnp.bfloat16)
        dn = (((1,), (1,)), ((), ()))
        acc = lax.dot_general(a1, w[:, :D_HALF], dn,
                              preferred_element_type=jnp.float32)
        acc = acc + lax.dot_general(a2, w[:, D_HALF:], dn,
                                    preferred_element_type=jnp.float32)
        o_ref[...] = jnp.maximum(acc + b_ref[0], 0.0)

    grid_spec = pltpu.PrefetchScalarGridSpec(
        num_scalar_prefetch=1,
        grid=(NB,),
        in_specs=[
            pl.BlockSpec((TM, D_HALF), lambda i, bt: (i, 0)),
            pl.BlockSpec((TM, D_HALF), lambda i, bt: (i, 0)),
            pl.BlockSpec((1, D_HALF, 2 * D_HALF), lambda i, bt: (bt[i], 0, 0)),
            pl.BlockSpec((1, 1, D_HALF), lambda i, bt: (bt[i], 0, 0)),
        ],
        out_specs=pl.BlockSpec((TM, D_HALF), lambda i, bt: (i, 0)),
    )
    return pl.pallas_call(
        mm_kernel,
        grid_spec=grid_spec,
        out_shape=jax.ShapeDtypeStruct((MP, D_HALF), jnp.float32),
    )(block_type, sx, se, W16, b.reshape(N_TYPES, 1, D_HALF))


def kernel(x, edge_attr, node_types, W, b):
    n = x.shape[0]
    t = node_types.astype(jnp.int32)

    nchunks = -(-n // _RANK_S)
    np2 = nchunks * _RANK_S
    t_pad = jnp.pad(t, (0, np2 - n), constant_values=N_TYPES)
    ohf = (t_pad[None, :] == jnp.arange(N_TYPES, dtype=jnp.int32)[:, None]
           ).astype(jnp.float32).reshape(N_TYPES, nchunks, _RANK_S)
    ar = jnp.arange(_RANK_S, dtype=jnp.int32)
    tri = (ar[:, None] < ar[None, :]).astype(jnp.float32)
    local_rank = lax.dot_general(
        ohf, tri, (((2,), (0,)), ((), ())),
        precision=lax.Precision.HIGHEST)
    chunk_cnt = ohf.sum(axis=2)
    chunk_base = jnp.cumsum(chunk_cnt, axis=1) - chunk_cnt
    counts = chunk_cnt.sum(axis=1).astype(jnp.int32)
    padded = ((counts + TM - 1) // TM) * TM
    pstart = (jnp.cumsum(padded) - padded).astype(jnp.float32)
    slot_f = ((local_rank + chunk_base[:, :, None]
               + pstart[:, None, None]) * ohf).sum(axis=0)
    slot = slot_f.reshape(np2).astype(jnp.int32)
    bend = (jnp.cumsum(padded) // TM).astype(jnp.int32)
    block_type = jnp.minimum(
        jnp.searchsorted(bend, jnp.arange(NB, dtype=jnp.int32), side="right"),
        N_TYPES - 1,
    ).astype(jnp.int32)

    sx, se = _dispatch(x, edge_attr, slot)

    out_sorted = _grouped_matmul(sx, se, block_type, W.astype(jnp.bfloat16), b)

    return _collect(out_sorted, slot, n)
